# baseline (speedup 1.0000x reference)
import functools

import jax
import jax.numpy as jnp
from jax import lax
from jax.experimental import pallas as pl
from jax.experimental.pallas import tpu as pltpu
from jax.experimental.pallas import tpu_sc as plsc

B, L = 4096, 200
D = 64
EOI = 256000
SCALE = float(D) ** 0.5

NC, NS, LANES = 2, 16, 16
NW = NC * NS
BB = B // NW


def _splat(vec, idx):
    return lax.gather(
        vec,
        jnp.full((LANES, 1), idx, jnp.int32),
        lax.GatherDimensionNumbers(
            offset_dims=(), collapsed_slice_dims=(0,), start_index_map=(0,)
        ),
        (1,),
        mode=lax.GatherScatterMode.PROMISE_IN_BOUNDS,
    )


def _sc_embed(table, ids2d, eoi):
    mesh = plsc.VectorSubcoreMesh(core_axis_name="c", subcore_axis_name="s")

    @functools.partial(
        pl.kernel,
        out_type=jax.ShapeDtypeStruct((L, D // 8, NW, 8, BB), jnp.float32),
        mesh=mesh,
        compiler_params=pltpu.CompilerParams(
            use_tc_tiling_on_sc=False, needs_layout_passes=False
        ),
        scratch_types=[
            pltpu.VMEM((L, BB), jnp.int32),
            pltpu.VMEM((BB, D), jnp.float32),
            pltpu.VMEM((BB, D), jnp.float32),
            pltpu.VMEM((D // 8, 8, BB), jnp.float32),
            pltpu.VMEM((D // 8, 8, BB), jnp.float32),
            pltpu.VMEM((D,), jnp.float32),
            pltpu.SemaphoreType.DMA,
            pltpu.SemaphoreType.DMA,
        ],
    )
    def body(table_hbm, ids_hbm, eoi_hbm, out_hbm, idx_v, g0, g1, t0, t1,
             eoi_v, sem0, sem1):
        wid = lax.axis_index("s") * NC + lax.axis_index("c")
        pltpu.sync_copy(ids_hbm.at[:, pl.ds(wid * BB, BB)], idx_v)
        pltpu.sync_copy(eoi_hbm, eoi_v)
        eoi_regs = [eoi_v[pl.ds(j * LANES, LANES)] for j in range(D // LANES)]
        iota = jnp.arange(LANES, dtype=jnp.int32)

        def gather(l, g, sem):
            return pltpu.make_async_copy(
                table_hbm.at[idx_v.at[l]], g, sem
            )

        def compute(l, g, t):
            def kbody(k, carry):
                iv = idx_v[l, pl.ds(k * LANES, LANES)]
                bvec = jnp.where(iv == EOI, 1.0, 0.0).astype(jnp.float32)
                avec = SCALE - SCALE * bvec
                rvec = iota + k * LANES
                bsl = pl.ds(k * LANES, LANES)
                for d in range(D):
                    es = _splat(eoi_regs[d // LANES], d % LANES)
                    val = plsc.load_gather(
                        g, [rvec, jnp.full((LANES,), d, jnp.int32)]
                    )
                    t[d // 8, d % 8, bsl] = val * avec + es * bvec
                return carry

            lax.fori_loop(0, BB // LANES, kbody, 0)

        gather(0, g0, sem0).start()

        def pair(ll, carry):
            l0 = ll * 2
            gather(l0 + 1, g1, sem1).start()
            gather(l0, g0, sem0).wait()
            compute(l0, g0, t0)
            pltpu.sync_copy(t0, out_hbm.at[l0, :, wid])

            @pl.when(ll + 1 < L // 2)
            def _():
                gather(l0 + 2, g0, sem0).start()

            gather(l0 + 1, g1, sem1).wait()
            compute(l0 + 1, g1, t1)
            pltpu.sync_copy(t1, out_hbm.at[l0 + 1, :, wid])
            return carry

        lax.fori_loop(0, L // 2, pair, 0)

    return body(table, ids2d, eoi)


def kernel(input_ids, embedding, eoi_embedding):
    outp = _sc_embed(embedding, input_ids.T, eoi_embedding)
    return outp.transpose(2, 4, 0, 1, 3).reshape(B, L, D)


# speedup vs baseline: 1.1253x; 1.1253x over previous
import functools

import jax
import jax.numpy as jnp
from jax import lax
from jax.experimental import pallas as pl
from jax.experimental.pallas import tpu as pltpu
from jax.experimental.pallas import tpu_sc as plsc

B, L = 4096, 200
D = 64
EOI = 256000
SCALE = float(D) ** 0.5

NC, NS, LANES = 2, 16, 16
NW = NC * NS
BB = B // NW


def _splat(vec, idx):
    return lax.gather(
        vec,
        jnp.full((LANES, 1), idx, jnp.int32),
        lax.GatherDimensionNumbers(
            offset_dims=(), collapsed_slice_dims=(0,), start_index_map=(0,)
        ),
        (1,),
        mode=lax.GatherScatterMode.PROMISE_IN_BOUNDS,
    )


def _sc_embed(table, ids2d, eoi):
    mesh = plsc.VectorSubcoreMesh(core_axis_name="c", subcore_axis_name="s")

    @functools.partial(
        pl.kernel,
        out_type=jax.ShapeDtypeStruct((L, D // 8, NW, 8, BB), jnp.float32),
        mesh=mesh,
        compiler_params=pltpu.CompilerParams(
            use_tc_tiling_on_sc=False, needs_layout_passes=False
        ),
        scratch_types=[
            pltpu.VMEM((L, BB), jnp.int32),
            pltpu.VMEM((BB, D), jnp.float32),
            pltpu.VMEM((BB, D), jnp.float32),
            pltpu.VMEM((D // 8, 8, BB), jnp.float32),
            pltpu.VMEM((D // 8, 8, BB), jnp.float32),
            pltpu.VMEM((D,), jnp.float32),
            pltpu.SemaphoreType.DMA,
            pltpu.SemaphoreType.DMA,
        ],
    )
    def body(table_hbm, ids_hbm, eoi_hbm, out_hbm, idx_v, g0, g1, t0, t1,
             eoi_v, sem0, sem1):
        wid = lax.axis_index("s") * NC + lax.axis_index("c")
        pltpu.sync_copy(ids_hbm.at[:, pl.ds(wid * BB, BB)], idx_v)
        pltpu.sync_copy(eoi_hbm, eoi_v)
        eoi_regs = [eoi_v[pl.ds(j * LANES, LANES)] for j in range(D // LANES)]
        iota = jnp.arange(LANES, dtype=jnp.int32)

        def gather(l, g, sem):
            return pltpu.make_async_copy(
                table_hbm.at[idx_v.at[l]], g, sem
            )

        dvals = [iota + j * LANES for j in range(D // LANES)]
        dhv = [dv // 8 for dv in dvals]
        dlv = [dv % 8 for dv in dvals]

        def compute(l, g, t):
            def rbody(r, carry):
                k0 = (r // LANES) * LANES
                ivw = idx_v[l, pl.ds(k0, LANES)]
                idsp = _splat(ivw, r - k0)
                bv = jnp.where(idsp == EOI, 1.0, 0.0).astype(jnp.float32)
                av = SCALE - SCALE * bv
                rfull = jnp.full((LANES,), r, jnp.int32)
                for j in range(D // LANES):
                    val = g[r, pl.ds(j * LANES, LANES)]
                    res = val * av + eoi_regs[j] * bv
                    plsc.store_scatter(t, [dhv[j], dlv[j], rfull], res)
                return carry

            lax.fori_loop(0, BB, rbody, 0)

        gather(0, g0, sem0).start()

        def pair(ll, carry):
            l0 = ll * 2
            gather(l0 + 1, g1, sem1).start()
            gather(l0, g0, sem0).wait()
            compute(l0, g0, t0)
            pltpu.sync_copy(t0, out_hbm.at[l0, :, wid])

            @pl.when(ll + 1 < L // 2)
            def _():
                gather(l0 + 2, g0, sem0).start()

            gather(l0 + 1, g1, sem1).wait()
            compute(l0 + 1, g1, t1)
            pltpu.sync_copy(t1, out_hbm.at[l0 + 1, :, wid])
            return carry

        lax.fori_loop(0, L // 2, pair, 0)

    return body(table, ids2d, eoi)


def kernel(input_ids, embedding, eoi_embedding):
    outp = _sc_embed(embedding, input_ids.T, eoi_embedding)
    return outp.transpose(2, 4, 0, 1, 3).reshape(B, L, D)


# speedup vs baseline: 1.6201x; 1.4397x over previous
import functools

import jax
import jax.numpy as jnp
from jax import lax
from jax.experimental import pallas as pl
from jax.experimental.pallas import tpu as pltpu
from jax.experimental.pallas import tpu_sc as plsc

D = 64
DP = 128
EOI = 256000
SCALE = float(D) ** 0.5

NC, NS, LANES = 2, 16, 16
NW = NC * NS
CHUNK = 512


def _splat(vec, idx):
    return lax.gather(
        vec,
        idx,
        lax.GatherDimensionNumbers(
            offset_dims=(), collapsed_slice_dims=(0,), start_index_map=(0,)
        ),
        (1,),
        mode=lax.GatherScatterMode.PROMISE_IN_BOUNDS,
    )


def _sc_embed(n_ids, table, ids, eoi):
    bpw = n_ids // NW
    nchunk = bpw // CHUNK
    mesh = plsc.VectorSubcoreMesh(core_axis_name="c", subcore_axis_name="s")

    @functools.partial(
        pl.kernel,
        out_type=jax.ShapeDtypeStruct((n_ids, D), jnp.float32),
        mesh=mesh,
        compiler_params=pltpu.CompilerParams(use_tc_tiling_on_sc=False),
        scratch_types=[
            pltpu.VMEM((bpw,), jnp.int32),
            pltpu.VMEM((CHUNK, DP), jnp.float32),
            pltpu.VMEM((D,), jnp.float32),
            pltpu.SemaphoreType.DMA,
        ],
    )
    def body(table_hbm, ids_hbm, eoi_hbm, out_hbm, idx_v, rows, eoi_v, sem):
        wid = lax.axis_index("s") * NC + lax.axis_index("c")
        base = wid * bpw
        pltpu.sync_copy(ids_hbm.at[pl.ds(base, bpw)], idx_v)
        pltpu.sync_copy(eoi_hbm, eoi_v)
        eoi_regs = [eoi_v[pl.ds(j * LANES, LANES)] for j in range(D // LANES)]

        def chunk_body(c, carry):
            cbase = c * CHUNK
            pltpu.async_copy(
                table_hbm.at[idx_v.at[pl.ds(cbase, CHUNK)]], rows, sem
            ).wait()

            def group(g, gcarry):
                iv = idx_v[pl.ds(cbase + g * LANES, LANES)]
                bvec = jnp.where(iv == EOI, 1.0, 0.0).astype(jnp.float32)
                avec = SCALE - SCALE * bvec
                for r in range(LANES):
                    row = g * LANES + r
                    rsel = jnp.full((LANES, 1), r, jnp.int32)
                    a = _splat(avec, rsel)
                    b = _splat(bvec, rsel)
                    for j in range(D // LANES):
                        sl = pl.ds(j * LANES, LANES)
                        rows[row, sl] = rows[row, sl] * a + eoi_regs[j] * b

                return gcarry

            lax.fori_loop(0, CHUNK // LANES, group, 0)
            pltpu.sync_copy(
                rows.at[:, pl.ds(0, D)],
                out_hbm.at[pl.ds(base + cbase, CHUNK)],
            )
            return carry

        lax.fori_loop(0, nchunk, chunk_body, 0)

    return body(table, ids, eoi)


def kernel(input_ids, embedding, eoi_embedding):
    tpad = jnp.pad(embedding, ((0, 0), (0, DP - D)))
    ids = input_ids.reshape(-1)
    out = _sc_embed(ids.shape[0], tpad, ids, eoi_embedding)
    return out.reshape(input_ids.shape + (D,))
